# Initial kernel scaffold; baseline (speedup 1.0000x reference)
import jax
import jax.numpy as jnp
from jax.experimental import pallas as pl


def kernel(phoneme_tensor, embedding_weight):
    raise NotImplementedError("write your pallas kernel here")


# speedup vs baseline: 2.0318x; 2.0318x over previous
import functools

import jax
import jax.numpy as jnp
from jax import lax
from jax.experimental import pallas as pl
from jax.experimental.pallas import tpu as pltpu
from jax.experimental.pallas import tpu_sc as plsc

_NC = 2
_NS = 16
_NW = _NC * _NS

_CH = 128
_K = 8
_BLK = _K * _CH


@functools.partial(jax.jit, static_argnums=(2, 3, 4))
def _gather_flat(idx2d, table, n_rows, per_w, n_blocks):
    d = table.shape[1]
    mesh = plsc.VectorSubcoreMesh(core_axis_name="c", subcore_axis_name="s")

    @functools.partial(
        pl.kernel,
        out_type=jax.ShapeDtypeStruct((n_rows, d), jnp.float32),
        mesh=mesh,
        scratch_types=[
            pltpu.VMEM((_K, _CH), jnp.int32),
            pltpu.VMEM((_BLK, d), jnp.float32),
            pltpu.SemaphoreType.DMA,
        ],
        compiler_params=pltpu.CompilerParams(use_tc_tiling_on_sc=False),
    )
    def emb(idx_hbm, table_hbm, out_hbm, idx_v, rows_v, sem):
        wid = lax.axis_index("s") * _NC + lax.axis_index("c")
        base = wid * per_w

        def body(b, carry):
            row0 = pl.multiple_of(base + b * _BLK, _BLK)
            pltpu.sync_copy(
                idx_hbm.at[pl.ds(pl.multiple_of(row0 // _CH, _K), _K)], idx_v
            )
            cps = [
                pltpu.async_copy(
                    table_hbm.at[idx_v.at[j]],
                    rows_v.at[pl.ds(j * _CH, _CH)],
                    sem,
                )
                for j in range(_K)
            ]
            for cp in cps:
                cp.wait()
            pltpu.sync_copy(rows_v, out_hbm.at[pl.ds(row0, _BLK)])
            return carry

        lax.fori_loop(0, n_blocks, body, 0)

    return emb(idx2d, table)


def kernel(phoneme_tensor, embedding_weight):
    b, l, c = phoneme_tensor.shape
    v, d = embedding_weight.shape
    n = b * l * c
    per_w = n // _NW
    n_blocks = per_w // _BLK
    assert per_w * _NW == n and n_blocks * _BLK == per_w
    idx2d = phoneme_tensor.astype(jnp.int32).reshape(n // _CH, _CH)
    out = _gather_flat(idx2d, embedding_weight, n, per_w, n_blocks)
    return out.reshape(b, l, c * d)


# speedup vs baseline: 2.0449x; 1.0065x over previous
import functools

import jax
import jax.numpy as jnp
from jax import lax
from jax.experimental import pallas as pl
from jax.experimental.pallas import tpu as pltpu
from jax.experimental.pallas import tpu_sc as plsc

_NC = 2
_NS = 16
_NW = _NC * _NS

_CH = 128
_K = 4
_BLK = _K * _CH
_NBUF = 2


@functools.partial(jax.jit, static_argnums=(2, 3, 4))
def _gather_flat(idx2d, table, n_rows, per_w, n_outer):
    d = table.shape[1]
    mesh = plsc.VectorSubcoreMesh(core_axis_name="c", subcore_axis_name="s")

    @functools.partial(
        pl.kernel,
        out_type=jax.ShapeDtypeStruct((n_rows, d), jnp.float32),
        mesh=mesh,
        scratch_types=[
            pltpu.VMEM((_NBUF * _K, _CH), jnp.int32),
            pltpu.VMEM((_NBUF, _BLK, d), jnp.float32),
            pltpu.SemaphoreType.DMA,
            pltpu.SemaphoreType.DMA,
            pltpu.SemaphoreType.DMA,
        ],
        compiler_params=pltpu.CompilerParams(use_tc_tiling_on_sc=False),
    )
    def emb(idx_hbm, table_hbm, out_hbm, idx_v, rows_v, gsem, wsem0, wsem1):
        wid = lax.axis_index("s") * _NC + lax.axis_index("c")
        base = wid * per_w
        wsems = (wsem0, wsem1)

        def body(i, carry):
            row0 = pl.multiple_of(base + i * (_NBUF * _BLK), _NBUF * _BLK)
            pltpu.sync_copy(
                idx_hbm.at[
                    pl.ds(pl.multiple_of(row0 // _CH, _NBUF * _K), _NBUF * _K)
                ],
                idx_v,
            )
            for p in range(_NBUF):
                rp = rows_v.at[p]
                out_slice = out_hbm.at[pl.ds(row0 + p * _BLK, _BLK)]
                @pl.when(i > 0)
                def _():
                    pltpu.make_async_copy(rp, out_slice, wsems[p]).wait()

                cps = [
                    pltpu.async_copy(
                        table_hbm.at[idx_v.at[p * _K + j]],
                        rp.at[pl.ds(j * _CH, _CH)],
                        gsem,
                    )
                    for j in range(_K)
                ]
                for cp in cps:
                    cp.wait()
                pltpu.async_copy(rp, out_slice, wsems[p])
            return carry

        lax.fori_loop(0, n_outer, body, 0)
        tail0 = (n_outer - 1) * (_NBUF * _BLK)
        for p in range(_NBUF):
            pltpu.make_async_copy(
                rows_v.at[p],
                out_hbm.at[pl.ds(base + tail0 + p * _BLK, _BLK)],
                wsems[p],
            ).wait()

    return emb(idx2d, table)


def kernel(phoneme_tensor, embedding_weight):
    b, l, c = phoneme_tensor.shape
    v, d = embedding_weight.shape
    n = b * l * c
    per_w = n // _NW
    n_outer = per_w // (_NBUF * _BLK)
    assert per_w * _NW == n and n_outer * _NBUF * _BLK == per_w
    idx2d = phoneme_tensor.astype(jnp.int32).reshape(n // _CH, _CH)
    out = _gather_flat(idx2d, embedding_weight, n, per_w, n_outer)
    return out.reshape(b, l, c * d)


# speedup vs baseline: 2.0564x; 1.0056x over previous
import functools

import jax
import jax.numpy as jnp
from jax import lax
from jax.experimental import pallas as pl
from jax.experimental.pallas import tpu as pltpu
from jax.experimental.pallas import tpu_sc as plsc

_NC = 2
_NS = 16
_NW = _NC * _NS

_CH = 128
_K = 6
_BLK = _K * _CH
_NBUF = 2


@functools.partial(jax.jit, static_argnums=(2, 3, 4))
def _gather_flat(idx2d, table, n_rows, per_w, n_outer):
    d = table.shape[1]
    mesh = plsc.VectorSubcoreMesh(core_axis_name="c", subcore_axis_name="s")

    @functools.partial(
        pl.kernel,
        out_type=jax.ShapeDtypeStruct((n_rows, d), jnp.float32),
        mesh=mesh,
        scratch_types=[
            pltpu.VMEM((_NBUF * _K, _CH), jnp.int32),
            pltpu.VMEM((_NBUF, _BLK, d), jnp.float32),
            pltpu.SemaphoreType.DMA,
            pltpu.SemaphoreType.DMA,
            pltpu.SemaphoreType.DMA,
        ],
        compiler_params=pltpu.CompilerParams(use_tc_tiling_on_sc=False),
    )
    def emb(idx_hbm, table_hbm, out_hbm, idx_v, rows_v, gsem, wsem0, wsem1):
        wid = lax.axis_index("s") * _NC + lax.axis_index("c")
        base = wid * per_w
        wsems = (wsem0, wsem1)

        def body(i, carry):
            row0 = pl.multiple_of(base + i * (_NBUF * _BLK), _NBUF * _BLK)
            pltpu.sync_copy(
                idx_hbm.at[
                    pl.ds(pl.multiple_of(row0 // _CH, _NBUF * _K), _NBUF * _K)
                ],
                idx_v,
            )
            all_cps = []
            for p in range(_NBUF):
                rp = rows_v.at[p]
                out_slice = out_hbm.at[pl.ds(row0 + p * _BLK, _BLK)]
                @pl.when(i > 0)
                def _():
                    pltpu.make_async_copy(rp, out_slice, wsems[p]).wait()

                all_cps.append(
                    [
                        pltpu.async_copy(
                            table_hbm.at[idx_v.at[p * _K + j]],
                            rp.at[pl.ds(j * _CH, _CH)],
                            gsem,
                        )
                        for j in range(_K)
                    ]
                )
            for p in range(_NBUF):
                for cp in all_cps[p]:
                    cp.wait()
                pltpu.async_copy(
                    rows_v.at[p],
                    out_hbm.at[pl.ds(row0 + p * _BLK, _BLK)],
                    wsems[p],
                )
            return carry

        lax.fori_loop(0, n_outer, body, 0)
        tail0 = (n_outer - 1) * (_NBUF * _BLK)
        for p in range(_NBUF):
            pltpu.make_async_copy(
                rows_v.at[p],
                out_hbm.at[pl.ds(base + tail0 + p * _BLK, _BLK)],
                wsems[p],
            ).wait()

    return emb(idx2d, table)


def kernel(phoneme_tensor, embedding_weight):
    b, l, c = phoneme_tensor.shape
    v, d = embedding_weight.shape
    n = b * l * c
    per_w = n // _NW
    n_outer = per_w // (_NBUF * _BLK)
    assert per_w * _NW == n and n_outer * _NBUF * _BLK == per_w
    idx2d = phoneme_tensor.astype(jnp.int32).reshape(n // _CH, _CH)
    out = _gather_flat(idx2d, embedding_weight, n, per_w, n_outer)
    return out.reshape(b, l, c * d)


# speedup vs baseline: 2.2179x; 1.0786x over previous
import functools

import jax
import jax.numpy as jnp
from jax import lax
from jax.experimental import pallas as pl
from jax.experimental.pallas import tpu as pltpu
from jax.experimental.pallas import tpu_sc as plsc

_NC = 2
_NS = 16
_NW = _NC * _NS

_LPB = 32
_RPB = 3 * _LPB
_NBUF = 4
_GRP = 32
_GIDX = _GRP * _RPB


@functools.partial(jax.jit, static_argnums=(2,))
def _gather_flat(idx, table_pad, n_rows):
    n_tok = n_rows // 3
    per_w_tok = n_tok // _NW
    n_groups = per_w_tok // (_GRP * _LPB)
    n_inner = _GRP // _NBUF
    mesh = plsc.VectorSubcoreMesh(core_axis_name="c", subcore_axis_name="s")

    @functools.partial(
        pl.kernel,
        out_type=jax.ShapeDtypeStruct((n_tok, 192), jnp.float32),
        mesh=mesh,
        scratch_types=[
            pltpu.VMEM((_GIDX,), jnp.int32),
            pltpu.VMEM((_NBUF, _RPB, 128), jnp.float32),
            pltpu.VMEM((_NBUF, _LPB, 192), jnp.float32),
            pltpu.SemaphoreType.DMA,
            pltpu.SemaphoreType.DMA,
            pltpu.SemaphoreType.DMA,
            pltpu.SemaphoreType.DMA,
            pltpu.SemaphoreType.DMA,
        ],
    )
    def emb(idx_hbm, table_hbm, out_hbm, idx_v, gbuf, pret, gsem, w0, w1, w2, w3):
        wid = lax.axis_index("s") * _NC + lax.axis_index("c")
        tok0 = wid * per_w_tok
        wsems = (w0, w1, w2, w3)

        def group(g, carry):
            gtok0 = tok0 + g * (_GRP * _LPB)
            pltpu.sync_copy(
                idx_hbm.at[pl.ds(pl.multiple_of(gtok0 * 3, _GIDX), _GIDX)],
                idx_v,
            )

            def inner(i2, carry2):
                first = (g == 0) & (i2 == 0)
                cps = []
                for p in range(_NBUF):
                    beta = i2 * _NBUF + p
                    cps.append(
                        pltpu.async_copy(
                            table_hbm.at[idx_v.at[pl.ds(beta * _RPB, _RPB)]],
                            gbuf.at[p],
                            gsem,
                        )
                    )
                for p in range(_NBUF):
                    beta = i2 * _NBUF + p
                    l0 = pl.multiple_of(
                        gtok0 + (i2 * _NBUF + p) * _LPB, _LPB
                    )
                    out_slice = out_hbm.at[pl.ds(l0, _LPB)]
                    cps[p].wait()
                    @pl.when(jnp.logical_not(first))
                    def _():
                        pltpu.make_async_copy(
                            pret.at[p], out_slice, wsems[p]
                        ).wait()
                    gp = gbuf.at[p]
                    pp = pret.at[p]
                    for r in range(_RPB):
                        dl = r // 3
                        dc = (r % 3) * 64
                        for k in range(4):
                            pp[dl, pl.ds(dc + k * 16, 16)] = gp[
                                r, pl.ds(k * 16, 16)
                            ]
                    pltpu.async_copy(pret.at[p], out_slice, wsems[p])
                return carry2

            lax.fori_loop(0, n_inner, inner, 0)
            return carry

        lax.fori_loop(0, n_groups, group, 0)
        tail = tok0 + (n_groups * _GRP - _NBUF) * _LPB
        for p in range(_NBUF):
            pltpu.make_async_copy(
                pret.at[p],
                out_hbm.at[pl.ds(tail + p * _LPB, _LPB)],
                wsems[p],
            ).wait()

    return emb(idx, table_pad)


def kernel(phoneme_tensor, embedding_weight):
    b, l, c = phoneme_tensor.shape
    v, d = embedding_weight.shape
    n = b * l * c
    assert c == 3 and d == 64
    per_w_tok = (n // 3) // _NW
    assert per_w_tok % (_GRP * _LPB) == 0
    idx = phoneme_tensor.astype(jnp.int32).reshape(n)
    table_pad = jnp.pad(embedding_weight, ((0, 0), (0, 128 - d)))
    out = _gather_flat(idx, table_pad, n)
    return out.reshape(b, l, c * d)
